# Initial kernel scaffold; baseline (speedup 1.0000x reference)
import jax
import jax.numpy as jnp
from jax.experimental import pallas as pl


def kernel(x, edge_index, batch, W_ml, b_ml, W_ih, W_hh, b_ih, b_hh, W_fc, b_fc):
    raise NotImplementedError("write your pallas kernel here")


# speedup vs baseline: 5.5592x; 5.5592x over previous
import functools

import jax
import jax.numpy as jnp
from jax import lax
from jax.experimental import pallas as pl
from jax.experimental.pallas import tpu as pltpu
from jax.experimental.pallas import tpu_sc as plsc

N = 10000
H = 128
HH = H // 2
E = 320000
G = 512
NP = 10240
CH = 128
C = -(-E // (16 * CH))
EPAD = 16 * C * CH
STRIPE = NP // 16

BLK = 512
NB = NP // BLK
IB = 1024


_sc_mesh = plsc.VectorSubcoreMesh(core_axis_name="c", subcore_axis_name="s")


@functools.partial(
    pl.kernel,
    out_type=[
        jax.ShapeDtypeStruct((2, NP, HH), jnp.float32),
        jax.ShapeDtypeStruct((NP, 16), jnp.float32),
    ],
    mesh=_sc_mesh,
    scratch_types=[
        pltpu.VMEM((C, CH), jnp.int32),
        pltpu.VMEM((C, CH), jnp.int32),
        pltpu.VMEM((CH, HH), jnp.float32),
        pltpu.VMEM((CH, 16), jnp.float32),
        pltpu.VMEM_SHARED((NP, HH), jnp.float32),
        pltpu.VMEM_SHARED((NP, 16), jnp.float32),
        pltpu.SemaphoreType.DMA,
    ],
    compiler_params=pltpu.CompilerParams(use_tc_tiling_on_sc=False),
)
def _sc_edges(x2_hbm, srcs_hbm, dsts_hbm, out_sum, out_deg,
              src_v, dst_v, rows_v, ones_v, acc_sh, deg_sh, sem):
    cid = lax.axis_index("c")
    sid = lax.axis_index("s")

    pltpu.sync_copy(srcs_hbm.at[cid, sid], src_v)
    pltpu.sync_copy(dsts_hbm.at[sid], dst_v)

    zeros16 = jnp.zeros((16,), jnp.float32)
    ones16 = jnp.full((16,), 1.0, jnp.float32)

    def _zrows(i, carry):
        r = i // (HH // 16)
        c = i % (HH // 16)
        rows_v[r, pl.ds(c * 16, 16)] = zeros16
        return carry

    lax.fori_loop(0, CH * HH // 16, _zrows, 0)

    def _fill(i, carry):
        ones_v[i, :] = zeros16
        return carry

    lax.fori_loop(0, CH, _fill, 0)
    for k in range(STRIPE // CH):
        pltpu.sync_copy(rows_v, acc_sh.at[pl.ds(sid * STRIPE + k * CH, CH)])
        pltpu.sync_copy(ones_v, deg_sh.at[pl.ds(sid * STRIPE + k * CH, CH)])

    def _fill2(i, carry):
        ones_v[i, :] = ones16
        return carry

    lax.fori_loop(0, CH, _fill2, 0)
    plsc.subcore_barrier()

    def _edge_chunk(j, carry):
        pltpu.async_copy(x2_hbm.at[src_v.at[j]], rows_v, sem).wait()
        pltpu.sync_copy(rows_v, acc_sh.at[dst_v.at[j]], add=True)

        @pl.when(cid == 0)
        def _():
            pltpu.sync_copy(ones_v, deg_sh.at[dst_v.at[j]], add=True)

        return carry

    lax.fori_loop(0, C, _edge_chunk, 0)
    plsc.subcore_barrier()

    pltpu.sync_copy(acc_sh.at[pl.ds(sid * STRIPE, STRIPE)],
                    out_sum.at[cid, pl.ds(sid * STRIPE, STRIPE)])

    @pl.when(cid == 0)
    def _():
        pltpu.sync_copy(deg_sh.at[pl.ds(sid * STRIPE, STRIPE)],
                        out_deg.at[pl.ds(sid * STRIPE, STRIPE)])


def _gru_pool_body(x_ref, xs_ref, degp_ref, b3_ref, wml_ref, bml_ref,
                   wih_ref, whh_ref, bih_ref, bhh_ref, gsum_ref, gcnt_ref):
    i = pl.program_id(0)
    x = x_ref[...]
    deg = jnp.maximum(degp_ref[:, 0], 1.0)
    xsum = jnp.concatenate([xs_ref[0], xs_ref[1]], axis=-1)
    xavg = xsum / deg[:, None]
    wml = wml_ref[...]
    bml = bml_ref[...]
    h0 = jnp.dot(x, wml, preferred_element_type=jnp.float32) + bml
    msg = jnp.dot(xavg, wml, preferred_element_type=jnp.float32) + bml
    gi = jnp.dot(msg, wih_ref[...], preferred_element_type=jnp.float32) + bih_ref[...]
    gh = jnp.dot(h0, whh_ref[...], preferred_element_type=jnp.float32) + bhh_ref[...]
    r = jax.nn.sigmoid(gi[:, :H] + gh[:, :H])
    z = jax.nn.sigmoid(gi[:, H:2 * H] + gh[:, H:2 * H])
    n = jnp.tanh(gi[:, 2 * H:] + r * gh[:, 2 * H:])
    h1 = (1.0 - z) * n + z * h0

    bid = b3_ref[0, 0, :]
    gids = lax.broadcasted_iota(jnp.int32, (G, BLK), 0)
    p = (gids == bid[None, :]).astype(jnp.float32)
    ps = jnp.dot(p, h1, preferred_element_type=jnp.float32)
    pc = jnp.sum(p, axis=1, keepdims=True)

    @pl.when(i == 0)
    def _():
        gsum_ref[...] = jnp.zeros_like(gsum_ref)
        gcnt_ref[...] = jnp.zeros_like(gcnt_ref)

    gsum_ref[...] += ps
    gcnt_ref[...] += jnp.broadcast_to(pc, (G, H))


_gru_pool = pl.pallas_call(
    _gru_pool_body,
    grid=(NB,),
    in_specs=[
        pl.BlockSpec((BLK, H), lambda i: (i, 0)),
        pl.BlockSpec((2, BLK, HH), lambda i: (0, i, 0)),
        pl.BlockSpec((BLK, 16), lambda i: (i, 0)),
        pl.BlockSpec((1, 1, BLK), lambda i: (i, 0, 0)),
        pl.BlockSpec((H, H), lambda i: (0, 0)),
        pl.BlockSpec((1, H), lambda i: (0, 0)),
        pl.BlockSpec((H, 3 * H), lambda i: (0, 0)),
        pl.BlockSpec((H, 3 * H), lambda i: (0, 0)),
        pl.BlockSpec((1, 3 * H), lambda i: (0, 0)),
        pl.BlockSpec((1, 3 * H), lambda i: (0, 0)),
    ],
    out_specs=[
        pl.BlockSpec((G, H), lambda i: (0, 0)),
        pl.BlockSpec((G, H), lambda i: (0, 0)),
    ],
    out_shape=[
        jax.ShapeDtypeStruct((G, H), jnp.float32),
        jax.ShapeDtypeStruct((G, H), jnp.float32),
    ],
)


def _fc_body(gsum_ref, gcnt_ref, wfct_ref, bfc_ref, out_ref):
    g = gsum_ref[...] / jnp.maximum(gcnt_ref[:, :1], 1.0)
    out_ref[...] = (jnp.dot(g, wfct_ref[...], preferred_element_type=jnp.float32)
                    + bfc_ref[...])


def _make_fc(ni):
    nblocks = -(-ni // IB)
    return pl.pallas_call(
        _fc_body,
        grid=(nblocks,),
        in_specs=[
            pl.BlockSpec((G, H), lambda j: (0, 0)),
            pl.BlockSpec((G, H), lambda j: (0, 0)),
            pl.BlockSpec((H, IB), lambda j: (0, j)),
            pl.BlockSpec((1, IB), lambda j: (0, j)),
        ],
        out_specs=pl.BlockSpec((G, IB), lambda j: (0, j)),
        out_shape=jax.ShapeDtypeStruct((G, ni), jnp.float32),
    )


def kernel(x, edge_index, batch, W_ml, b_ml, W_ih, W_hh, b_ih, b_hh, W_fc, b_fc):
    ni = W_fc.shape[0]
    src = edge_index[0].astype(jnp.int32)
    dst = edge_index[1].astype(jnp.int32)
    pad = jnp.full((EPAD - E,), N, jnp.int32)
    srcs = jnp.concatenate([src, pad]).reshape(16, C, CH)
    srcs2 = jnp.stack([srcs, srcs + NP])
    dsts = jnp.concatenate([dst, pad]).reshape(16, C, CH)
    xpad = jnp.concatenate([x, jnp.zeros((NP - N, H), x.dtype)], axis=0)
    x2 = jnp.concatenate([xpad[:, :HH], xpad[:, HH:]], axis=0)
    batch3 = jnp.concatenate(
        [batch.astype(jnp.int32), jnp.full((NP - N,), G, jnp.int32)]
    ).reshape(NB, 1, BLK)

    xsum, deg = _sc_edges(x2, srcs2, dsts)

    gsum, gcnt = _gru_pool(
        xpad, xsum, deg, batch3,
        W_ml.T, b_ml.reshape(1, H),
        W_ih.T, W_hh.T, b_ih.reshape(1, 3 * H), b_hh.reshape(1, 3 * H),
    )
    scores = _make_fc(ni)(gsum, gcnt, W_fc.T, b_fc.reshape(1, ni))
    return scores


# speedup vs baseline: 6.7273x; 1.2101x over previous
import functools

import jax
import jax.numpy as jnp
from jax import lax
from jax.experimental import pallas as pl
from jax.experimental.pallas import tpu as pltpu
from jax.experimental.pallas import tpu_sc as plsc

N = 10000
H = 128
HH = H // 2
E = 320000
G = 512
NP = 10240
CH = 128
C = 158
EPAD = 16 * C * CH
STRIPE = NP // 16

BLK = 512
NB = NP // BLK
IB = 1024


_sc_mesh = plsc.VectorSubcoreMesh(core_axis_name="c", subcore_axis_name="s")


@functools.partial(
    pl.kernel,
    out_type=[
        jax.ShapeDtypeStruct((2, NP, HH), jnp.float32),
        jax.ShapeDtypeStruct((2, NP, 16), jnp.float32),
    ],
    mesh=_sc_mesh,
    scratch_types=[
        pltpu.VMEM((C, CH), jnp.int32),
        pltpu.VMEM((C, CH), jnp.int32),
        pltpu.VMEM((CH, HH), jnp.float32),
        pltpu.VMEM((CH, HH), jnp.float32),
        pltpu.VMEM((CH, 16), jnp.float32),
        pltpu.VMEM_SHARED((NP, HH), jnp.float32),
        pltpu.VMEM_SHARED((NP, 16), jnp.float32),
        pltpu.SemaphoreType.DMA,
        pltpu.SemaphoreType.DMA,
        pltpu.SemaphoreType.DMA,
        pltpu.SemaphoreType.DMA,
        pltpu.SemaphoreType.DMA,
    ],
    compiler_params=pltpu.CompilerParams(use_tc_tiling_on_sc=False),
)
def _sc_edges(x2_hbm, srcs_hbm, dsts_hbm, out_sum, out_deg,
              src_v, dst_v, rows_a, rows_b, ones_v, acc_sh, deg_sh,
              sem_ga, sem_gb, sem_sa, sem_sb, sem_d):
    cid = lax.axis_index("c")
    sid = lax.axis_index("s")

    pltpu.sync_copy(srcs_hbm.at[cid, sid], src_v)
    pltpu.sync_copy(dsts_hbm.at[sid], dst_v)

    zeros16 = jnp.zeros((16,), jnp.float32)
    ones16 = jnp.full((16,), 1.0, jnp.float32)

    def _zrows(i, carry):
        r = i // (HH // 16)
        c = i % (HH // 16)
        rows_a[r, pl.ds(c * 16, 16)] = zeros16
        return carry

    lax.fori_loop(0, CH * HH // 16, _zrows, 0)

    def _fill(i, carry):
        ones_v[i, :] = zeros16
        return carry

    lax.fori_loop(0, CH, _fill, 0)
    for k in range(STRIPE // CH):
        pltpu.sync_copy(rows_a, acc_sh.at[pl.ds(sid * STRIPE + k * CH, CH)])
        pltpu.sync_copy(ones_v, deg_sh.at[pl.ds(sid * STRIPE + k * CH, CH)])

    def _fill2(i, carry):
        ones_v[i, :] = ones16
        return carry

    lax.fori_loop(0, CH, _fill2, 0)
    plsc.subcore_barrier()

    pltpu.async_copy(x2_hbm.at[src_v.at[0]], rows_a, sem_ga)
    pltpu.async_copy(x2_hbm.at[src_v.at[1]], rows_b, sem_gb)

    @pl.loop(0, C, step=2)
    def _chunks(j):
        pltpu.make_async_copy(x2_hbm.at[src_v.at[j]], rows_a, sem_ga).wait()
        pltpu.async_copy(rows_a, acc_sh.at[dst_v.at[j]], sem_sa, add=True)

        @pl.when(cid == 0)
        def _():
            @pl.when(j > 0)
            def _():
                pltpu.make_async_copy(ones_v, deg_sh.at[dst_v.at[j]], sem_d).wait()

            pltpu.async_copy(ones_v, deg_sh.at[dst_v.at[j]], sem_d, add=True)

        pltpu.make_async_copy(x2_hbm.at[src_v.at[j]], rows_b, sem_gb).wait()
        pltpu.async_copy(rows_b, acc_sh.at[dst_v.at[j + 1]], sem_sb, add=True)

        @pl.when(cid == 1)
        def _():
            @pl.when(j > 0)
            def _():
                pltpu.make_async_copy(ones_v, deg_sh.at[dst_v.at[j]], sem_d).wait()

            pltpu.async_copy(ones_v, deg_sh.at[dst_v.at[j + 1]], sem_d, add=True)

        pltpu.make_async_copy(rows_a, acc_sh.at[dst_v.at[j]], sem_sa).wait()

        @pl.when(j + 2 < C)
        def _():
            pltpu.async_copy(x2_hbm.at[src_v.at[j + 2]], rows_a, sem_ga)

        pltpu.make_async_copy(rows_b, acc_sh.at[dst_v.at[j]], sem_sb).wait()

        @pl.when(j + 3 < C)
        def _():
            pltpu.async_copy(x2_hbm.at[src_v.at[j + 3]], rows_b, sem_gb)

    pltpu.make_async_copy(ones_v, deg_sh.at[dst_v.at[0]], sem_d).wait()
    plsc.subcore_barrier()

    pltpu.sync_copy(acc_sh.at[pl.ds(sid * STRIPE, STRIPE)],
                    out_sum.at[cid, pl.ds(sid * STRIPE, STRIPE)])
    pltpu.sync_copy(deg_sh.at[pl.ds(sid * STRIPE, STRIPE)],
                    out_deg.at[cid, pl.ds(sid * STRIPE, STRIPE)])


def _gru_pool_body(x_ref, xs_ref, degp_ref, b3_ref, wml_ref, bml_ref,
                   wih_ref, whh_ref, bih_ref, bhh_ref, gsum_ref, gcnt_ref):
    i = pl.program_id(0)
    x = x_ref[...]
    deg = jnp.maximum(degp_ref[0, :, 0] + degp_ref[1, :, 0], 1.0)
    xsum = jnp.concatenate([xs_ref[0], xs_ref[1]], axis=-1)
    xavg = xsum / deg[:, None]
    wml = wml_ref[...]
    bml = bml_ref[...]
    h0 = jnp.dot(x, wml, preferred_element_type=jnp.float32) + bml
    msg = jnp.dot(xavg, wml, preferred_element_type=jnp.float32) + bml
    gi = jnp.dot(msg, wih_ref[...], preferred_element_type=jnp.float32) + bih_ref[...]
    gh = jnp.dot(h0, whh_ref[...], preferred_element_type=jnp.float32) + bhh_ref[...]
    r = jax.nn.sigmoid(gi[:, :H] + gh[:, :H])
    z = jax.nn.sigmoid(gi[:, H:2 * H] + gh[:, H:2 * H])
    n = jnp.tanh(gi[:, 2 * H:] + r * gh[:, 2 * H:])
    h1 = (1.0 - z) * n + z * h0

    bid = b3_ref[0, 0, :]
    gids = lax.broadcasted_iota(jnp.int32, (G, BLK), 0)
    p = (gids == bid[None, :]).astype(jnp.float32)
    ps = jnp.dot(p, h1, preferred_element_type=jnp.float32)
    pc = jnp.sum(p, axis=1, keepdims=True)

    @pl.when(i == 0)
    def _():
        gsum_ref[...] = jnp.zeros_like(gsum_ref)
        gcnt_ref[...] = jnp.zeros_like(gcnt_ref)

    gsum_ref[...] += ps
    gcnt_ref[...] += jnp.broadcast_to(pc, (G, H))


_gru_pool = pl.pallas_call(
    _gru_pool_body,
    grid=(NB,),
    in_specs=[
        pl.BlockSpec((BLK, H), lambda i: (i, 0)),
        pl.BlockSpec((2, BLK, HH), lambda i: (0, i, 0)),
        pl.BlockSpec((2, BLK, 16), lambda i: (0, i, 0)),
        pl.BlockSpec((1, 1, BLK), lambda i: (i, 0, 0)),
        pl.BlockSpec((H, H), lambda i: (0, 0)),
        pl.BlockSpec((1, H), lambda i: (0, 0)),
        pl.BlockSpec((H, 3 * H), lambda i: (0, 0)),
        pl.BlockSpec((H, 3 * H), lambda i: (0, 0)),
        pl.BlockSpec((1, 3 * H), lambda i: (0, 0)),
        pl.BlockSpec((1, 3 * H), lambda i: (0, 0)),
    ],
    out_specs=[
        pl.BlockSpec((G, H), lambda i: (0, 0)),
        pl.BlockSpec((G, H), lambda i: (0, 0)),
    ],
    out_shape=[
        jax.ShapeDtypeStruct((G, H), jnp.float32),
        jax.ShapeDtypeStruct((G, H), jnp.float32),
    ],
)


def _fc_body(gsum_ref, gcnt_ref, wfct_ref, bfc_ref, out_ref):
    g = gsum_ref[...] / jnp.maximum(gcnt_ref[:, :1], 1.0)
    out_ref[...] = (jnp.dot(g, wfct_ref[...], preferred_element_type=jnp.float32)
                    + bfc_ref[...])


def _make_fc(ni):
    nblocks = -(-ni // IB)
    return pl.pallas_call(
        _fc_body,
        grid=(nblocks,),
        in_specs=[
            pl.BlockSpec((G, H), lambda j: (0, 0)),
            pl.BlockSpec((G, H), lambda j: (0, 0)),
            pl.BlockSpec((H, IB), lambda j: (0, j)),
            pl.BlockSpec((1, IB), lambda j: (0, j)),
        ],
        out_specs=pl.BlockSpec((G, IB), lambda j: (0, j)),
        out_shape=jax.ShapeDtypeStruct((G, ni), jnp.float32),
    )


def kernel(x, edge_index, batch, W_ml, b_ml, W_ih, W_hh, b_ih, b_hh, W_fc, b_fc):
    ni = W_fc.shape[0]
    src = edge_index[0].astype(jnp.int32)
    dst = edge_index[1].astype(jnp.int32)
    pad = jnp.full((EPAD - E,), N, jnp.int32)
    srcs = jnp.concatenate([src, pad]).reshape(16, C, CH)
    srcs2 = jnp.stack([srcs, srcs + NP])
    dsts = jnp.concatenate([dst, pad]).reshape(16, C, CH)
    xpad = jnp.concatenate([x, jnp.zeros((NP - N, H), x.dtype)], axis=0)
    x2 = jnp.concatenate([xpad[:, :HH], xpad[:, HH:]], axis=0)
    batch3 = jnp.concatenate(
        [batch.astype(jnp.int32), jnp.full((NP - N,), G, jnp.int32)]
    ).reshape(NB, 1, BLK)

    xsum, deg = _sc_edges(x2, srcs2, dsts)

    gsum, gcnt = _gru_pool(
        xpad, xsum, deg, batch3,
        W_ml.T, b_ml.reshape(1, H),
        W_ih.T, W_hh.T, b_ih.reshape(1, 3 * H), b_hh.reshape(1, 3 * H),
    )
    scores = _make_fc(ni)(gsum, gcnt, W_fc.T, b_fc.reshape(1, ni))
    return scores
